# baseline (speedup 1.0000x reference)
import functools

import jax
import jax.numpy as jnp
from jax import lax
from jax.experimental import pallas as pl
from jax.experimental.pallas import tpu as pltpu
from jax.experimental.pallas import tpu_sc as plsc

_NC = 2
_NS = 16
_NW = _NC * _NS
_L = 16

_D = 2048
_C = 16


def _permute_rows(x2, perm32):
    R, D = x2.shape
    rows_per_w = R // _NW
    n_chunks = rows_per_w // _C
    groups = D // _L

    mesh = plsc.VectorSubcoreMesh(
        core_axis_name="c", subcore_axis_name="s",
        num_cores=_NC, num_subcores=_NS)

    @functools.partial(
        pl.kernel,
        mesh=mesh,
        out_type=jax.ShapeDtypeStruct((R, D), jnp.float32),
        scratch_types=[
            pltpu.VMEM((D,), jnp.int32),
            pltpu.VMEM((_C, D), jnp.float32),
            pltpu.VMEM((_C, D), jnp.float32),
        ],
        compiler_params=pltpu.CompilerParams(needs_layout_passes=False),
    )
    def k(x_hbm, perm_hbm, out_hbm, idx_v, in_v, out_v):
        wid = lax.axis_index("s") * _NC + lax.axis_index("c")
        base = wid * rows_per_w
        pltpu.sync_copy(perm_hbm, idx_v)

        @pl.loop(0, n_chunks)
        def _chunk(c):
            row0 = base + c * _C
            pltpu.sync_copy(x_hbm.at[pl.ds(row0, _C)], in_v)

            @pl.loop(0, groups)
            def _group(j):
                col = j * _L
                idx = idx_v[pl.ds(col, _L)]
                for r in range(_C):
                    row = jnp.full((_L,), r, dtype=jnp.int32)
                    vals = plsc.load_gather(in_v, [row, idx])
                    out_v[r, pl.ds(col, _L)] = vals

            pltpu.sync_copy(out_v, out_hbm.at[pl.ds(row0, _C)])

    return k(x2, perm32)


def kernel(x, perm):
    B, S, D = x.shape
    x2 = x.reshape(B * S, D)
    out2 = _permute_rows(x2, perm.astype(jnp.int32))
    out = out2.reshape(B, S, D)
    log_det = jnp.zeros((B, S), dtype=x.dtype)
    return (out, log_det)


# speedup vs baseline: 1.2091x; 1.2091x over previous
import functools

import jax
import jax.numpy as jnp
from jax import lax
from jax.experimental import pallas as pl
from jax.experimental.pallas import tpu as pltpu
from jax.experimental.pallas import tpu_sc as plsc

_NC = 2
_NS = 16
_NW = _NC * _NS
_L = 16

_D = 2048
_C = 8


def _permute_rows(x2, perm32):
    R, D = x2.shape
    rows_per_w = R // _NW
    n_chunks = rows_per_w // _C
    groups = D // _L

    mesh = plsc.VectorSubcoreMesh(
        core_axis_name="c", subcore_axis_name="s",
        num_cores=_NC, num_subcores=_NS)

    @functools.partial(
        pl.kernel,
        mesh=mesh,
        out_type=jax.ShapeDtypeStruct((R, D), jnp.float32),
        scratch_types=[
            pltpu.VMEM((D,), jnp.int32),
            pltpu.VMEM((_C, D), jnp.float32),
            pltpu.VMEM((_C, D), jnp.float32),
            pltpu.VMEM((_C, D), jnp.float32),
            pltpu.VMEM((_C, D), jnp.float32),
            pltpu.SemaphoreType.DMA,
            pltpu.SemaphoreType.DMA,
            pltpu.SemaphoreType.DMA,
            pltpu.SemaphoreType.DMA,
        ],
        compiler_params=pltpu.CompilerParams(needs_layout_passes=False),
    )
    def k(x_hbm, perm_hbm, out_hbm, idx_v,
          in0, in1, out0, out1, si0, si1, so0, so1):
        ins, outs = (in0, in1), (out0, out1)
        sis, sos = (si0, si1), (so0, so1)
        wid = lax.axis_index("s") * _NC + lax.axis_index("c")
        base = wid * rows_per_w
        pltpu.sync_copy(perm_hbm, idx_v)

        def start_in(c, b):
            pltpu.async_copy(x_hbm.at[pl.ds(base + c * _C, _C)], ins[b], sis[b])

        def wait_in(b):
            pltpu.make_async_copy(x_hbm.at[pl.ds(0, _C)], ins[b], sis[b]).wait()

        def start_out(c, b):
            pltpu.async_copy(outs[b], out_hbm.at[pl.ds(base + c * _C, _C)], sos[b])

        def wait_out(b):
            pltpu.make_async_copy(outs[b], out_hbm.at[pl.ds(0, _C)], sos[b]).wait()

        start_in(0, 0)

        @pl.loop(0, n_chunks, step=2)
        def _pair(c0):
            for b in range(2):
                c = c0 + b

                @pl.when(c + 1 < n_chunks)
                def _prefetch():
                    start_in(c + 1, 1 - b)

                wait_in(b)

                @pl.when(c >= 2)
                def _drain():
                    wait_out(b)

                @pl.loop(0, groups, unroll=2)
                def _group(j):
                    col = j * _L
                    idx = idx_v[pl.ds(col, _L)]
                    for r in range(_C):
                        row = jnp.full((_L,), r, dtype=jnp.int32)
                        vals = plsc.load_gather(ins[b], [row, idx])
                        outs[b][r, pl.ds(col, _L)] = vals

                start_out(c, b)

        wait_out(0)
        wait_out(1)

    return k(x2, perm32)


def kernel(x, perm):
    B, S, D = x.shape
    x2 = x.reshape(B * S, D)
    out2 = _permute_rows(x2, perm.astype(jnp.int32))
    out = out2.reshape(B, S, D)
    log_det = jnp.zeros((B, S), dtype=x.dtype)
    return (out, log_det)


# speedup vs baseline: 3.6424x; 3.0125x over previous
import functools

import jax
import jax.numpy as jnp
from jax import lax
from jax.experimental import pallas as pl
from jax.experimental.pallas import tpu as pltpu
from jax.experimental.pallas import tpu_sc as plsc

_NC = 2
_NS = 16
_NW = _NC * _NS
_L = 16

_D = 2048
_C = 8


def _permute_rows(x2, perm32):
    R, D = x2.shape
    rows_per_w = R // _NW
    n_chunks = rows_per_w // _C
    groups = D // _L

    mesh = plsc.VectorSubcoreMesh(
        core_axis_name="c", subcore_axis_name="s",
        num_cores=_NC, num_subcores=_NS)

    @functools.partial(
        pl.kernel,
        mesh=mesh,
        out_type=jax.ShapeDtypeStruct((R, D), jnp.float32),
        scratch_types=[
            pltpu.VMEM((D,), jnp.int32),
            pltpu.VMEM((_C, D), jnp.float32),
            pltpu.VMEM((_C, D), jnp.float32),
            pltpu.VMEM((_C, D), jnp.float32),
            pltpu.VMEM((_C, D), jnp.float32),
            pltpu.SemaphoreType.DMA,
            pltpu.SemaphoreType.DMA,
            pltpu.SemaphoreType.DMA,
            pltpu.SemaphoreType.DMA,
        ],
        compiler_params=pltpu.CompilerParams(needs_layout_passes=False),
    )
    def k(x_hbm, perm_hbm, out_hbm, idx_v,
          in0, in1, out0, out1, si0, si1, so0, so1):
        ins, outs = (in0, in1), (out0, out1)
        sis, sos = (si0, si1), (so0, so1)
        wid = lax.axis_index("s") * _NC + lax.axis_index("c")
        base = wid * rows_per_w
        pltpu.sync_copy(perm_hbm, idx_v)

        def start_in(c, b):
            pltpu.async_copy(x_hbm.at[pl.ds(base + c * _C, _C)], ins[b], sis[b])

        def wait_in(b):
            pltpu.make_async_copy(x_hbm.at[pl.ds(0, _C)], ins[b], sis[b]).wait()

        def start_out(c, b):
            pltpu.async_copy(outs[b], out_hbm.at[pl.ds(base + c * _C, _C)], sos[b])

        def wait_out(b):
            pltpu.make_async_copy(outs[b], out_hbm.at[pl.ds(0, _C)], sos[b]).wait()

        start_in(0, 0)

        @pl.loop(0, n_chunks, step=2)
        def _pair(c0):
            for b in range(2):
                c = c0 + b

                @pl.when(c + 1 < n_chunks)
                def _prefetch():
                    start_in(c + 1, 1 - b)

                wait_in(b)

                @pl.when(c >= 2)
                def _drain():
                    wait_out(b)

                @plsc.parallel_loop(0, groups, unroll=4)
                def _group(j):
                    col = j * _L
                    idx = idx_v[pl.ds(col, _L)]
                    for r in range(_C):
                        row = jnp.full((_L,), r, dtype=jnp.int32)
                        vals = plsc.load_gather(ins[b], [row, idx])
                        outs[b][r, pl.ds(col, _L)] = vals

                start_out(c, b)

        wait_out(0)
        wait_out(1)

    return k(x2, perm32)


def kernel(x, perm):
    B, S, D = x.shape
    x2 = x.reshape(B * S, D)
    out2 = _permute_rows(x2, perm.astype(jnp.int32))
    out = out2.reshape(B, S, D)
    log_det = jnp.zeros((B, S), dtype=x.dtype)
    return (out, log_det)


# speedup vs baseline: 3.6498x; 1.0020x over previous
import functools

import jax
import jax.numpy as jnp
from jax import lax
from jax.experimental import pallas as pl
from jax.experimental.pallas import tpu as pltpu
from jax.experimental.pallas import tpu_sc as plsc

_NC = 2
_NS = 16
_NW = _NC * _NS
_L = 16

_D = 2048
_C = 8


def _permute_rows(x2, perm32):
    R, D = x2.shape
    rows_per_w = R // _NW
    n_chunks = rows_per_w // _C
    groups = D // _L

    mesh = plsc.VectorSubcoreMesh(
        core_axis_name="c", subcore_axis_name="s",
        num_cores=_NC, num_subcores=_NS)

    @functools.partial(
        pl.kernel,
        mesh=mesh,
        out_type=jax.ShapeDtypeStruct((R, D), jnp.float32),
        scratch_types=[
            pltpu.VMEM((D,), jnp.int32),
            pltpu.VMEM((_C, D), jnp.float32),
            pltpu.VMEM((_C, D), jnp.float32),
            pltpu.VMEM((_C, D), jnp.float32),
            pltpu.VMEM((_C, D), jnp.float32),
            pltpu.SemaphoreType.DMA,
            pltpu.SemaphoreType.DMA,
            pltpu.SemaphoreType.DMA,
            pltpu.SemaphoreType.DMA,
        ],
        compiler_params=pltpu.CompilerParams(needs_layout_passes=False),
    )
    def k(x_hbm, perm_hbm, out_hbm, idx_v,
          in0, in1, out0, out1, si0, si1, so0, so1):
        ins, outs = (in0, in1), (out0, out1)
        sis, sos = (si0, si1), (so0, so1)
        wid = lax.axis_index("s") * _NC + lax.axis_index("c")
        base = wid * rows_per_w
        pltpu.sync_copy(perm_hbm, idx_v)

        def start_in(c, b):
            pltpu.async_copy(x_hbm.at[pl.ds(base + c * _C, _C)], ins[b], sis[b])

        def wait_in(b):
            pltpu.make_async_copy(x_hbm.at[pl.ds(0, _C)], ins[b], sis[b]).wait()

        def start_out(c, b):
            pltpu.async_copy(outs[b], out_hbm.at[pl.ds(base + c * _C, _C)], sos[b])

        def wait_out(b):
            pltpu.make_async_copy(outs[b], out_hbm.at[pl.ds(0, _C)], sos[b]).wait()

        start_in(0, 0)

        @pl.loop(0, n_chunks, step=2)
        def _pair(c0):
            for b in range(2):
                c = c0 + b

                @pl.when(c + 1 < n_chunks)
                def _prefetch():
                    start_in(c + 1, 1 - b)

                wait_in(b)

                @pl.when(c >= 2)
                def _drain():
                    wait_out(b)

                @plsc.parallel_loop(0, groups, unroll=8)
                def _group(j):
                    col = j * _L
                    idx = idx_v[pl.ds(col, _L)]
                    for r in range(_C):
                        row = jnp.full((_L,), r, dtype=jnp.int32)
                        vals = plsc.load_gather(ins[b], [row, idx])
                        outs[b][r, pl.ds(col, _L)] = vals

                start_out(c, b)

        wait_out(0)
        wait_out(1)

    return k(x2, perm32)


def kernel(x, perm):
    B, S, D = x.shape
    x2 = x.reshape(B * S, D)
    out2 = _permute_rows(x2, perm.astype(jnp.int32))
    out = out2.reshape(B, S, D)
    log_det = jnp.zeros((B, S), dtype=x.dtype)
    return (out, log_det)


# speedup vs baseline: 3.7647x; 1.0315x over previous
import functools

import jax
import jax.numpy as jnp
from jax import lax
from jax.experimental import pallas as pl
from jax.experimental.pallas import tpu as pltpu
from jax.experimental.pallas import tpu_sc as plsc

_NC = 2
_NS = 16
_NW = _NC * _NS
_L = 16

_D = 2048
_C = 4
_NBUF = 4


def _permute_rows(x2, perm32):
    R, D = x2.shape
    rows_per_w = R // _NW
    n_chunks = rows_per_w // _C
    assert n_chunks % _NBUF == 0
    groups = D // _L

    mesh = plsc.VectorSubcoreMesh(
        core_axis_name="c", subcore_axis_name="s",
        num_cores=_NC, num_subcores=_NS)

    @functools.partial(
        pl.kernel,
        mesh=mesh,
        out_type=jax.ShapeDtypeStruct((R, D), jnp.float32),
        scratch_types=[
            pltpu.VMEM((D,), jnp.int32),
            [pltpu.VMEM((_C, D), jnp.float32)] * _NBUF,
            [pltpu.VMEM((_C, D), jnp.float32)] * _NBUF,
            [pltpu.SemaphoreType.DMA] * _NBUF,
            [pltpu.SemaphoreType.DMA] * _NBUF,
        ],
        compiler_params=pltpu.CompilerParams(needs_layout_passes=False),
    )
    def k(x_hbm, perm_hbm, out_hbm, idx_v, ins, outs, sis, sos):
        wid = lax.axis_index("s") * _NC + lax.axis_index("c")
        base = wid * rows_per_w
        pltpu.sync_copy(perm_hbm, idx_v)

        def start_in(c, b):
            pltpu.async_copy(x_hbm.at[pl.ds(base + c * _C, _C)], ins[b], sis[b])

        def wait_in(b):
            pltpu.make_async_copy(x_hbm.at[pl.ds(0, _C)], ins[b], sis[b]).wait()

        def start_out(c, b):
            pltpu.async_copy(outs[b], out_hbm.at[pl.ds(base + c * _C, _C)], sos[b])

        def wait_out(b):
            pltpu.make_async_copy(outs[b], out_hbm.at[pl.ds(0, _C)], sos[b]).wait()

        for b in range(_NBUF - 1):
            start_in(b, b)

        @pl.loop(0, n_chunks, step=_NBUF)
        def _ring(c0):
            for b in range(_NBUF):
                c = c0 + b

                @pl.when(c + _NBUF - 1 < n_chunks)
                def _prefetch():
                    start_in(c + _NBUF - 1, (b + _NBUF - 1) % _NBUF)

                wait_in(b)

                @pl.when(c >= _NBUF)
                def _drain():
                    wait_out(b)

                @plsc.parallel_loop(0, groups, unroll=4)
                def _group(j):
                    col = j * _L
                    idx = idx_v[pl.ds(col, _L)]
                    for r in range(_C):
                        row = jnp.full((_L,), r, dtype=jnp.int32)
                        vals = plsc.load_gather(ins[b], [row, idx])
                        outs[b][r, pl.ds(col, _L)] = vals

                start_out(c, b)

        for b in range(_NBUF):
            wait_out(b)

    return k(x2, perm32)


def kernel(x, perm):
    B, S, D = x.shape
    x2 = x.reshape(B * S, D)
    out2 = _permute_rows(x2, perm.astype(jnp.int32))
    out = out2.reshape(B, S, D)
    log_det = jnp.zeros((B, S), dtype=x.dtype)
    return (out, log_det)
